# baseline (speedup 1.0000x reference)
import functools

import jax
import jax.numpy as jnp
from jax import lax
from jax.experimental import pallas as pl
from jax.experimental.pallas import tpu as pltpu
from jax.experimental.pallas import tpu_sc as plsc

D = 128
NPOS, NPC, NCOL = 64, 6, 2
NFUSED = NPOS * NPC * NCOL

B, L = 16384, 32
TOK = B * L
NC, NS = 2, 16
NW = NC * NS
TPW = TOK // NW
CHUNK = 128
NCHUNK = TPW // CHUNK
ROWS = TOK // D
RPW = TPW // D
NBUF = 5
LAG = 3
LANES = 16


def _sc_body(pos_hbm, pc_hbm, col_hbm, fidx_hbm, out_hbm,
             tab_sh, fidx_v, rows, pos_v, pcrow_v, colrow_v,
             semg, semo):
    sid = lax.axis_index("s")
    wid = sid * NC + lax.axis_index("c")
    tok0 = wid * TPW
    row0 = wid * RPW

    cp_f = pltpu.async_copy(fidx_hbm.at[pl.ds(row0, RPW)], fidx_v, semg[0])

    @pl.when(sid < NPC * NCOL)
    def _build():
        m = sid
        pltpu.sync_copy(pos_hbm, pos_v)
        pltpu.sync_copy(pc_hbm.at[pl.ds(m // NCOL, 1)], pcrow_v)
        pltpu.sync_copy(col_hbm.at[pl.ds(m % NCOL, 1)], colrow_v)
        for k in range(D // LANES):
            s = pl.ds(k * LANES, LANES)
            pcrow_v[0, s] = pcrow_v[0, s] + colrow_v[0, s]

        def build_row(r, carry):
            for k in range(D // LANES):
                s = pl.ds(k * LANES, LANES)
                pos_v[r, s] = pos_v[r, s] + pcrow_v[0, s]
            return carry

        lax.fori_loop(0, NPOS, build_row, None)
        pltpu.sync_copy(pos_v, tab_sh.at[pl.ds(m * NPOS, NPOS)])

    cp_f.wait()
    plsc.subcore_barrier()

    def gather(g, b):
        pltpu.async_copy(tab_sh.at[fidx_v.at[g]], rows[b], semg[b])

    def put(g, b):
        pltpu.async_copy(rows[b], out_hbm.at[pl.ds(tok0 + g * CHUNK, CHUNK)],
                         semo[b])

    def wait_gather(g, b):
        pltpu.make_async_copy(tab_sh.at[fidx_v.at[g]], rows[b], semg[b]).wait()

    def wait_put(g, b):
        pltpu.make_async_copy(rows[b],
                              out_hbm.at[pl.ds(tok0 + g * CHUNK, CHUNK)],
                              semo[b]).wait()

    for g in range(LAG):
        gather(g, g)

    for b in range(NBUF):
        if b >= 2:
            wait_put(b - 2, b - 2)
        gather(b + LAG, (b + LAG) % NBUF)
        wait_gather(b, b)
        put(b, b)

    def step(h, carry):
        for b in range(NBUF):
            g = h * NBUF + b
            bl = (b + LAG) % NBUF
            wait_put(g - 2, bl)
            gather(g + LAG, bl)
            wait_gather(g, b)
            put(g, b)
        return carry

    nsteady = (NCHUNK - LAG) // NBUF
    lax.fori_loop(1, nsteady, step, None)

    for g in range(NBUF * nsteady, NCHUNK):
        b = g % NBUF
        wait_gather(g, b)
        put(g, b)

    for g in range(NCHUNK - NBUF, NCHUNK):
        wait_put(g, g % NBUF)


def _sc_embed(pos, pc, col, fidx2):
    mesh = plsc.VectorSubcoreMesh(core_axis_name="c", subcore_axis_name="s")
    f = functools.partial(
        pl.kernel,
        out_type=jax.ShapeDtypeStruct((TOK, D), jnp.float32),
        mesh=mesh,
        scratch_types=[
            pltpu.VMEM_SHARED((NFUSED, D), jnp.float32),
            pltpu.VMEM((RPW, D), jnp.int32),
            [pltpu.VMEM((CHUNK, D), jnp.float32) for _ in range(NBUF)],
            pltpu.VMEM((NPOS, D), jnp.float32),
            pltpu.VMEM((1, D), jnp.float32),
            pltpu.VMEM((1, D), jnp.float32),
            [pltpu.SemaphoreType.DMA for _ in range(NBUF)],
            [pltpu.SemaphoreType.DMA for _ in range(NBUF)],
        ],
    )(_sc_body)
    return f(pos, pc, col, fidx2)


def kernel(pieces_ids, color_ids, indexes, position_emb, piece_emb, color_emb):
    fidx = (pieces_ids.astype(jnp.int32) * NCOL
            + color_ids.astype(jnp.int32)) * NPOS + indexes.astype(jnp.int32)
    out = _sc_embed(position_emb, piece_emb, color_emb, fidx.reshape(ROWS, D))
    return out.reshape(B, L, D)
